# baseline (speedup 1.0000x reference)
import jax
import jax.numpy as jnp
from jax import lax
from jax.experimental import pallas as pl
from jax.experimental.pallas import tpu as pltpu
from jax.experimental.pallas import tpu_sc as plsc

N_REAL = 100000
N_PAD = 100096
DUMMY = N_REAL
E_REAL = 1600000
E_PAD = 1638400
NCHUNK = 6400
C = 256
H = 16
RB = 128
NDC = 782
NTILES = 16
STRIPE = 6256


def _body(src_hbm, dst_hbm, emb_hbm, out_hbm, z_hbm, dis_hbm,
          A_sh,
          sidx, didx, rowb, atile, btile, dbuf,
          gsem, ssem, ssem2):
    cid = lax.axis_index("c")
    sid = lax.axis_index("s")

    def _zb(i, c):
        btile[i] = jnp.zeros((16,), jnp.float32)
        return c
    lax.fori_loop(0, RB, _zb, 0)

    def _zero_A():
        def _za(k, c):
            pltpu.sync_copy(btile, A_sh.at[pl.ds(sid * STRIPE + k * RB, RB)])
            return c
        lax.fori_loop(0, 48, _za, 0)
        pltpu.sync_copy(btile.at[pl.ds(0, 112)],
                        A_sh.at[pl.ds(sid * STRIPE + 48 * RB, 112)])

    _zero_A()

    def _fo(i, c):
        rowb[0, i] = jnp.full((16,), 1.0, jnp.float32)
        return c
    lax.fori_loop(0, C, _fo, 0)
    plsc.subcore_barrier()

    ebase = sid * 400
    def _deg_stage(st, c):
        pltpu.sync_copy(dst_hbm.at[pl.ds(ebase + st * 8, 8)], didx)
        hs = []
        for j in range(8):
            hs.append(pltpu.async_copy(rowb.at[0], A_sh.at[didx.at[j]],
                                       ssem, add=True))
        for h in hs:
            h.wait()
        return c
    lax.fori_loop(0, 50, _deg_stage, 0)
    plsc.subcore_barrier()

    nck = 48 + jnp.where(sid < NDC - 48 * NTILES, 1, 0)

    def _init_chunk(k, c):
        r0 = (sid + k * 16) * RB
        pltpu.sync_copy(A_sh.at[pl.ds(r0, RB)], atile)
        pltpu.sync_copy(emb_hbm.at[cid, pl.ds(r0, RB)], btile)
        def _row(i, cc):
            x = atile[i]
            xi = lax.bitcast_convert_type(x, jnp.int32)
            hh = jnp.int32(0x5F3759DF) - (xi >> 1)
            y = lax.bitcast_convert_type(hh, jnp.float32)
            for _ in range(4):
                y = y * (1.5 - 0.5 * x * y * y)
            sp = jnp.where(x > 0.5, y, 0.0)
            e = btile[i]
            dbuf[i] = sp
            atile[i] = sp * e
            btile[i] = 0.25 * e
            return cc
        lax.fori_loop(0, RB, _row, 0)
        pltpu.sync_copy(dbuf, dis_hbm.at[cid, pl.ds(r0, RB)])
        pltpu.sync_copy(atile, z_hbm.at[cid, pl.ds(r0, RB)])
        pltpu.sync_copy(btile, out_hbm.at[cid, pl.ds(r0, RB)])
        return c
    lax.fori_loop(0, nck, _init_chunk, 0)
    plsc.subcore_barrier()

    zc = z_hbm.at[cid]

    for layer in (1, 2, 3):
        last = layer == 3

        def _zb2(i, c):
            btile[i] = jnp.zeros((16,), jnp.float32)
            return c
        lax.fori_loop(0, RB, _zb2, 0)
        _zero_A()
        plsc.subcore_barrier()

        def _stage(st, c):
            pltpu.sync_copy(src_hbm.at[pl.ds(ebase + st * 8, 8)], sidx)
            pltpu.sync_copy(dst_hbm.at[pl.ds(ebase + st * 8, 8)], didx)

            gh = pltpu.async_copy(zc.at[sidx.at[0]], rowb.at[0], gsem)
            sh_prev = None
            for g in range(8):
                p = g % 2
                gh.wait()
                sh = pltpu.async_copy(rowb.at[p], A_sh.at[didx.at[g]],
                                      ssem if p == 0 else ssem2, add=True)
                if sh_prev is not None:
                    sh_prev.wait()
                if g < 7:
                    gh = pltpu.async_copy(zc.at[sidx.at[g + 1]],
                                          rowb.at[1 - p], gsem)
                sh_prev = sh
            sh_prev.wait()
            return c
        lax.fori_loop(0, 50, _stage, 0)
        plsc.subcore_barrier()

        def _dchunk(k, c):
            r0 = (sid + k * 16) * RB
            pltpu.sync_copy(A_sh.at[pl.ds(r0, RB)], atile)
            pltpu.sync_copy(dis_hbm.at[cid, pl.ds(r0, RB)], dbuf)
            pltpu.sync_copy(out_hbm.at[cid, pl.ds(r0, RB)], btile)
            def _row(i, cc):
                sp = dbuf[i]
                cur = sp * atile[i]
                btile[i] = btile[i] + 0.25 * cur
                if not last:
                    dbuf[i] = sp * cur
                return cc
            lax.fori_loop(0, RB, _row, 0)
            pltpu.sync_copy(btile, out_hbm.at[cid, pl.ds(r0, RB)])
            if not last:
                pltpu.sync_copy(dbuf, z_hbm.at[cid, pl.ds(r0, RB)])
            return c
        lax.fori_loop(0, nck, _dchunk, 0)
        plsc.subcore_barrier()


@jax.jit
def _gnn(src2, dst2, embs):
    mesh = plsc.VectorSubcoreMesh(core_axis_name="c", subcore_axis_name="s")
    f = pl.kernel(
        _body,
        out_type=(
            jax.ShapeDtypeStruct((2, N_PAD, H), jnp.float32),
            jax.ShapeDtypeStruct((2, N_PAD, H), jnp.float32),
            jax.ShapeDtypeStruct((2, N_PAD, H), jnp.float32),
        ),
        mesh=mesh,
        compiler_params=pltpu.CompilerParams(
            needs_layout_passes=False, use_tc_tiling_on_sc=False),
        scratch_types=(
            pltpu.VMEM_SHARED((N_PAD, H), jnp.float32),
            pltpu.VMEM((8, C), jnp.int32),
            pltpu.VMEM((8, C), jnp.int32),
            pltpu.VMEM((2, C, H), jnp.float32),
            pltpu.VMEM((RB, H), jnp.float32),
            pltpu.VMEM((RB, H), jnp.float32),
            pltpu.VMEM((RB, H), jnp.float32),
            pltpu.SemaphoreType.DMA,
            pltpu.SemaphoreType.DMA,
            pltpu.SemaphoreType.DMA,
        ),
    )
    return f(src2, dst2, embs)


def kernel(edge_index, edge_attrs, emb_weight):
    del edge_attrs
    src = edge_index[0]
    dst = edge_index[1]
    pad = jnp.full((E_PAD - E_REAL,), DUMMY, jnp.int32)
    src2 = jnp.concatenate([src, pad]).reshape(NCHUNK, C)
    dst2 = jnp.concatenate([dst, pad]).reshape(NCHUNK, C)
    embp = jnp.pad(emb_weight, ((0, N_PAD - N_REAL), (0, 0)))
    embs = jnp.stack([embp[:, :H], embp[:, H:]])
    out2, _, _ = _gnn(src2, dst2, embs)
    out = jnp.concatenate([out2[0, :N_REAL], out2[1, :N_REAL]], axis=1)
    return (emb_weight, out)


# speedup vs baseline: 1.1595x; 1.1595x over previous
import jax
import jax.numpy as jnp
from jax import lax
from jax.experimental import pallas as pl
from jax.experimental.pallas import tpu as pltpu
from jax.experimental.pallas import tpu_sc as plsc

N_REAL = 100000
N_PAD = 100096
DUMMY = N_REAL
E_REAL = 1600000
E_PAD = 1638400
NCHUNK = 12800
C = 128
H = 16
RB = 128
NDC = 782
NTILES = 16
STRIPE = 6256


def _body(src_hbm, dst_hbm, emb_hbm, out_hbm, z_hbm, dis_hbm,
          A_sh,
          sidx, didx, rowb, atile, btile, dbuf,
          gsem, ssem, ssem2, stsem):
    cid = lax.axis_index("c")
    sid = lax.axis_index("s")

    def _zb(i, c):
        btile[i] = jnp.zeros((16,), jnp.float32)
        return c
    lax.fori_loop(0, RB, _zb, 0)

    def _zero_A():
        def _za(k, c):
            pltpu.sync_copy(btile, A_sh.at[pl.ds(sid * STRIPE + k * RB, RB)])
            return c
        lax.fori_loop(0, 48, _za, 0)
        pltpu.sync_copy(btile.at[pl.ds(0, 112)],
                        A_sh.at[pl.ds(sid * STRIPE + 48 * RB, 112)])

    _zero_A()

    def _fo(i, c):
        rowb[0, i] = jnp.full((16,), 1.0, jnp.float32)
        return c
    lax.fori_loop(0, C, _fo, 0)
    plsc.subcore_barrier()

    ebase = sid * 800
    def _deg_stage(st, c):
        pltpu.sync_copy(dst_hbm.at[pl.ds(ebase + st * 16, 16)], didx.at[0])
        hs = []
        for j in range(16):
            hs.append(pltpu.async_copy(rowb.at[0], A_sh.at[didx.at[0, j]],
                                       ssem, add=True))
        for h in hs:
            h.wait()
        return c
    lax.fori_loop(0, 50, _deg_stage, 0)
    plsc.subcore_barrier()

    nck = 48 + jnp.where(sid < NDC - 48 * NTILES, 1, 0)

    def _init_chunk(k, c):
        r0 = (sid + k * 16) * RB
        pltpu.sync_copy(A_sh.at[pl.ds(r0, RB)], atile)
        pltpu.sync_copy(emb_hbm.at[cid, pl.ds(r0, RB)], btile)
        def _row(i, cc):
            x = atile[i]
            xi = lax.bitcast_convert_type(x, jnp.int32)
            hh = jnp.int32(0x5F3759DF) - (xi >> 1)
            y = lax.bitcast_convert_type(hh, jnp.float32)
            for _ in range(4):
                y = y * (1.5 - 0.5 * x * y * y)
            sp = jnp.where(x > 0.5, y, 0.0)
            e = btile[i]
            dbuf[i] = sp
            atile[i] = sp * e
            btile[i] = 0.25 * e
            return cc
        lax.fori_loop(0, RB, _row, 0)
        pltpu.sync_copy(dbuf, dis_hbm.at[cid, pl.ds(r0, RB)])
        pltpu.sync_copy(atile, z_hbm.at[cid, pl.ds(r0, RB)])
        pltpu.sync_copy(btile, out_hbm.at[cid, pl.ds(r0, RB)])
        return c
    lax.fori_loop(0, nck, _init_chunk, 0)
    plsc.subcore_barrier()

    zc = z_hbm.at[cid]

    for layer in (1, 2, 3):
        last = layer == 3

        def _zb2(i, c):
            btile[i] = jnp.zeros((16,), jnp.float32)
            return c
        lax.fori_loop(0, RB, _zb2, 0)
        _zero_A()
        plsc.subcore_barrier()

        pltpu.async_copy(src_hbm.at[pl.ds(ebase, 16)], sidx.at[0], stsem)
        pltpu.async_copy(dst_hbm.at[pl.ds(ebase, 16)], didx.at[0], stsem)

        def _stage(st, c):
            p = lax.rem(st, 2)
            pltpu.make_async_copy(src_hbm.at[pl.ds(ebase + st * 16, 16)],
                                  sidx.at[p], stsem).wait()
            pltpu.make_async_copy(dst_hbm.at[pl.ds(ebase + st * 16, 16)],
                                  didx.at[p], stsem).wait()

            @pl.when(st < 49)
            def _prefetch():
                pltpu.async_copy(src_hbm.at[pl.ds(ebase + (st + 1) * 16, 16)],
                                 sidx.at[1 - p], stsem)
                pltpu.async_copy(dst_hbm.at[pl.ds(ebase + (st + 1) * 16, 16)],
                                 didx.at[1 - p], stsem)

            def fire_g(g):
                bk = (g % 2) * 4
                return [pltpu.async_copy(zc.at[sidx.at[p, g * 4 + b]],
                                         rowb.at[bk + b], gsem)
                        for b in range(4)]

            def fire_s(g):
                bk = (g % 2) * 4
                return [pltpu.async_copy(rowb.at[bk + b],
                                         A_sh.at[didx.at[p, g * 4 + b]],
                                         ssem if g % 2 == 0 else ssem2,
                                         add=True)
                        for b in range(4)]

            gh = fire_g(0)
            sh_prev = None
            for g in range(4):
                for h in gh:
                    h.wait()
                sh = fire_s(g)
                if sh_prev is not None:
                    for h in sh_prev:
                        h.wait()
                if g < 3:
                    gh = fire_g(g + 1)
                sh_prev = sh
            for h in sh_prev:
                h.wait()
            return c
        lax.fori_loop(0, 50, _stage, 0)
        plsc.subcore_barrier()

        def _dchunk(k, c):
            r0 = (sid + k * 16) * RB
            pltpu.sync_copy(A_sh.at[pl.ds(r0, RB)], atile)
            pltpu.sync_copy(dis_hbm.at[cid, pl.ds(r0, RB)], dbuf)
            pltpu.sync_copy(out_hbm.at[cid, pl.ds(r0, RB)], btile)
            def _row(i, cc):
                sp = dbuf[i]
                cur = sp * atile[i]
                btile[i] = btile[i] + 0.25 * cur
                if not last:
                    dbuf[i] = sp * cur
                return cc
            lax.fori_loop(0, RB, _row, 0)
            pltpu.sync_copy(btile, out_hbm.at[cid, pl.ds(r0, RB)])
            if not last:
                pltpu.sync_copy(dbuf, z_hbm.at[cid, pl.ds(r0, RB)])
            return c
        lax.fori_loop(0, nck, _dchunk, 0)
        plsc.subcore_barrier()


@jax.jit
def _gnn(src2, dst2, embs):
    mesh = plsc.VectorSubcoreMesh(core_axis_name="c", subcore_axis_name="s")
    f = pl.kernel(
        _body,
        out_type=(
            jax.ShapeDtypeStruct((2, N_PAD, H), jnp.float32),
            jax.ShapeDtypeStruct((2, N_PAD, H), jnp.float32),
            jax.ShapeDtypeStruct((2, N_PAD, H), jnp.float32),
        ),
        mesh=mesh,
        compiler_params=pltpu.CompilerParams(
            needs_layout_passes=False, use_tc_tiling_on_sc=False),
        scratch_types=(
            pltpu.VMEM_SHARED((N_PAD, H), jnp.float32),
            pltpu.VMEM((2, 16, C), jnp.int32),
            pltpu.VMEM((2, 16, C), jnp.int32),
            pltpu.VMEM((8, C, H), jnp.float32),
            pltpu.VMEM((RB, H), jnp.float32),
            pltpu.VMEM((RB, H), jnp.float32),
            pltpu.VMEM((RB, H), jnp.float32),
            pltpu.SemaphoreType.DMA,
            pltpu.SemaphoreType.DMA,
            pltpu.SemaphoreType.DMA,
            pltpu.SemaphoreType.DMA,
        ),
    )
    return f(src2, dst2, embs)


def kernel(edge_index, edge_attrs, emb_weight):
    del edge_attrs
    src = edge_index[0]
    dst = edge_index[1]
    pad = jnp.full((E_PAD - E_REAL,), DUMMY, jnp.int32)
    src2 = jnp.concatenate([src, pad]).reshape(NCHUNK, C)
    dst2 = jnp.concatenate([dst, pad]).reshape(NCHUNK, C)
    embp = jnp.pad(emb_weight, ((0, N_PAD - N_REAL), (0, 0)))
    embs = jnp.stack([embp[:, :H], embp[:, H:]])
    out2, _, _ = _gnn(src2, dst2, embs)
    out = jnp.concatenate([out2[0, :N_REAL], out2[1, :N_REAL]], axis=1)
    return (emb_weight, out)


# speedup vs baseline: 1.1606x; 1.0010x over previous
import jax
import jax.numpy as jnp
from jax import lax
from jax.experimental import pallas as pl
from jax.experimental.pallas import tpu as pltpu
from jax.experimental.pallas import tpu_sc as plsc

N_REAL = 100000
N_PAD = 100096
DUMMY = N_REAL
E_REAL = 1600000
E_PAD = 1638400
NCHUNK = 12800
C = 128
H = 16
RB = 128
NDC = 782
NTILES = 16
STRIPE = 6256


def _body(src_hbm, dst_hbm, emb_hbm, out_hbm, z_hbm, dis_hbm,
          A_sh,
          sidx, didx, rowb, atile, btile, dbuf,
          gsem, ssem, ssem2, stsem):
    cid = lax.axis_index("c")
    sid = lax.axis_index("s")

    def _zb(i, c):
        btile[i] = jnp.zeros((16,), jnp.float32)
        return c
    lax.fori_loop(0, RB, _zb, 0)

    def _zero_A():
        def _za(k, c):
            pltpu.sync_copy(btile, A_sh.at[pl.ds(sid * STRIPE + k * RB, RB)])
            return c
        lax.fori_loop(0, 48, _za, 0)
        pltpu.sync_copy(btile.at[pl.ds(0, 112)],
                        A_sh.at[pl.ds(sid * STRIPE + 48 * RB, 112)])

    _zero_A()

    def _fo(i, c):
        rowb[0, i] = jnp.full((16,), 1.0, jnp.float32)
        return c
    lax.fori_loop(0, C, _fo, 0)
    plsc.subcore_barrier()

    ebase = sid * 800
    def _deg_stage(st, c):
        pltpu.sync_copy(dst_hbm.at[pl.ds(ebase + st * 16, 16)], didx.at[0])
        hs = []
        for j in range(16):
            hs.append(pltpu.async_copy(rowb.at[0], A_sh.at[didx.at[0, j]],
                                       ssem, add=True))
        for h in hs:
            h.wait()
        return c
    lax.fori_loop(0, 50, _deg_stage, 0)
    plsc.subcore_barrier()

    nck = 48 + jnp.where(sid < NDC - 48 * NTILES, 1, 0)

    def _init_chunk(k, c):
        r0 = (sid + k * 16) * RB
        pltpu.sync_copy(A_sh.at[pl.ds(r0, RB)], atile)
        pltpu.sync_copy(emb_hbm.at[cid, pl.ds(r0, RB)], btile)
        def _row(i, cc):
            x = atile[i]
            xi = lax.bitcast_convert_type(x, jnp.int32)
            hh = jnp.int32(0x5F3759DF) - (xi >> 1)
            y = lax.bitcast_convert_type(hh, jnp.float32)
            for _ in range(4):
                y = y * (1.5 - 0.5 * x * y * y)
            sp = jnp.where(x > 0.5, y, 0.0)
            e = btile[i]
            dbuf[i] = sp
            atile[i] = sp * e
            btile[i] = 0.25 * e
            return cc
        lax.fori_loop(0, RB, _row, 0)
        pltpu.sync_copy(dbuf, dis_hbm.at[cid, pl.ds(r0, RB)])
        pltpu.sync_copy(atile, z_hbm.at[cid, pl.ds(r0, RB)])
        pltpu.sync_copy(btile, out_hbm.at[cid, pl.ds(r0, RB)])
        return c
    lax.fori_loop(0, nck, _init_chunk, 0)
    plsc.subcore_barrier()

    zc = z_hbm.at[cid]

    for layer in (1, 2, 3):
        last = layer == 3

        if layer == 1:
            def _zb2(i, c):
                btile[i] = jnp.zeros((16,), jnp.float32)
                return c
            lax.fori_loop(0, RB, _zb2, 0)
            _zero_A()
            plsc.subcore_barrier()

        pltpu.async_copy(src_hbm.at[pl.ds(ebase, 16)], sidx.at[0], stsem)
        pltpu.async_copy(dst_hbm.at[pl.ds(ebase, 16)], didx.at[0], stsem)

        def _stage(st, c):
            p = lax.rem(st, 2)
            pltpu.make_async_copy(src_hbm.at[pl.ds(ebase + st * 16, 16)],
                                  sidx.at[p], stsem).wait()
            pltpu.make_async_copy(dst_hbm.at[pl.ds(ebase + st * 16, 16)],
                                  didx.at[p], stsem).wait()

            @pl.when(st < 49)
            def _prefetch():
                pltpu.async_copy(src_hbm.at[pl.ds(ebase + (st + 1) * 16, 16)],
                                 sidx.at[1 - p], stsem)
                pltpu.async_copy(dst_hbm.at[pl.ds(ebase + (st + 1) * 16, 16)],
                                 didx.at[1 - p], stsem)

            def fire_g(g):
                bk = (g % 2) * 4
                return [pltpu.async_copy(zc.at[sidx.at[p, g * 4 + b]],
                                         rowb.at[bk + b], gsem)
                        for b in range(4)]

            def fire_s(g):
                bk = (g % 2) * 4
                return [pltpu.async_copy(rowb.at[bk + b],
                                         A_sh.at[didx.at[p, g * 4 + b]],
                                         ssem if g % 2 == 0 else ssem2,
                                         add=True)
                        for b in range(4)]

            gh = fire_g(0)
            sh_prev = None
            for g in range(4):
                for h in gh:
                    h.wait()
                sh = fire_s(g)
                if sh_prev is not None:
                    for h in sh_prev:
                        h.wait()
                if g < 3:
                    gh = fire_g(g + 1)
                sh_prev = sh
            for h in sh_prev:
                h.wait()
            return c
        lax.fori_loop(0, 50, _stage, 0)
        plsc.subcore_barrier()

        def _dchunk(k, c):
            r0 = (sid + k * 16) * RB
            pltpu.sync_copy(A_sh.at[pl.ds(r0, RB)], atile)
            pltpu.sync_copy(dis_hbm.at[cid, pl.ds(r0, RB)], dbuf)
            pltpu.sync_copy(out_hbm.at[cid, pl.ds(r0, RB)], btile)
            def _row(i, cc):
                sp = dbuf[i]
                cur = sp * atile[i]
                btile[i] = btile[i] + 0.25 * cur
                if not last:
                    dbuf[i] = sp * cur
                    atile[i] = jnp.zeros((16,), jnp.float32)
                return cc
            lax.fori_loop(0, RB, _row, 0)
            pltpu.sync_copy(btile, out_hbm.at[cid, pl.ds(r0, RB)])
            if not last:
                pltpu.sync_copy(dbuf, z_hbm.at[cid, pl.ds(r0, RB)])
                pltpu.sync_copy(atile, A_sh.at[pl.ds(r0, RB)])
            return c
        lax.fori_loop(0, nck, _dchunk, 0)
        plsc.subcore_barrier()


@jax.jit
def _gnn(src2, dst2, embs):
    mesh = plsc.VectorSubcoreMesh(core_axis_name="c", subcore_axis_name="s")
    f = pl.kernel(
        _body,
        out_type=(
            jax.ShapeDtypeStruct((2, N_PAD, H), jnp.float32),
            jax.ShapeDtypeStruct((2, N_PAD, H), jnp.float32),
            jax.ShapeDtypeStruct((2, N_PAD, H), jnp.float32),
        ),
        mesh=mesh,
        compiler_params=pltpu.CompilerParams(
            needs_layout_passes=False, use_tc_tiling_on_sc=False),
        scratch_types=(
            pltpu.VMEM_SHARED((N_PAD, H), jnp.float32),
            pltpu.VMEM((2, 16, C), jnp.int32),
            pltpu.VMEM((2, 16, C), jnp.int32),
            pltpu.VMEM((8, C, H), jnp.float32),
            pltpu.VMEM((RB, H), jnp.float32),
            pltpu.VMEM((RB, H), jnp.float32),
            pltpu.VMEM((RB, H), jnp.float32),
            pltpu.SemaphoreType.DMA,
            pltpu.SemaphoreType.DMA,
            pltpu.SemaphoreType.DMA,
            pltpu.SemaphoreType.DMA,
        ),
    )
    return f(src2, dst2, embs)


def kernel(edge_index, edge_attrs, emb_weight):
    del edge_attrs
    src = edge_index[0]
    dst = edge_index[1]
    pad = jnp.full((E_PAD - E_REAL,), DUMMY, jnp.int32)
    src2 = jnp.concatenate([src, pad]).reshape(NCHUNK, C)
    dst2 = jnp.concatenate([dst, pad]).reshape(NCHUNK, C)
    embp = jnp.pad(emb_weight, ((0, N_PAD - N_REAL), (0, 0)))
    embs = jnp.stack([embp[:, :H], embp[:, H:]])
    out2, _, _ = _gnn(src2, dst2, embs)
    out = jnp.concatenate([out2[0, :N_REAL], out2[1, :N_REAL]], axis=1)
    return (emb_weight, out)
